# baseline (speedup 1.0000x reference)
import jax
import jax.numpy as jnp
from jax.experimental import pallas as pl
from jax.experimental.pallas import tpu as pltpu

_D_MODEL = 1024
_D_FFN = 4096
_TOP_K = 256
_BLK = 256


def _ffn_kernel(x_ref, wg_ref, wu_ref, wd_ref, o_ref):
    x = x_ref[...]
    g = jnp.dot(x, wg_ref[...], preferred_element_type=jnp.float32)
    u = jnp.dot(x.astype(jnp.bfloat16), wu_ref[...],
                preferred_element_type=jnp.float32)

    bits = jax.lax.bitcast_convert_type(g, jnp.int32)
    key = bits ^ ((bits >> 31) & jnp.int32(0x7FFFFFFF))

    def k2v(k):
        return jax.lax.bitcast_convert_type(
            k ^ ((k >> 31) & jnp.int32(0x7FFFFFFF)), jnp.float32)

    def v2k(v):
        b = jax.lax.bitcast_convert_type(v, jnp.int32)
        return b ^ ((b >> 31) & jnp.int32(0x7FFFFFFF))

    kf = jnp.float32(_TOP_K)
    lo = v2k(jnp.min(g, axis=1, keepdims=True))
    hi = v2k(jnp.max(g, axis=1, keepdims=True))
    cl = jnp.full_like(lo, g.shape[1]).astype(jnp.float32)
    ch = jnp.ones_like(lo).astype(jnp.float32)
    wl = jnp.ones_like(cl)
    wh = jnp.ones_like(cl)

    def active_rows(s):
        lo, hi, cl, ch, wl, wh, it = s
        w = hi - lo
        done = (cl == kf) | ((w >= 0) & (w <= 1))
        return ~done

    def cond(s):
        return jnp.any(active_rows(s)) & (s[-1] < 48)

    def body(s):
        lo, hi, cl, ch, wl, wh, it = s
        act = active_rows(s)
        a = (cl - kf) * wl
        bwt = (kf - ch) * wh
        frac = a / jnp.maximum(a + bwt, jnp.float32(1e-30))
        lv = k2v(lo)
        hv = k2v(hi)
        tgt = lv + (hv - lv) * frac
        cand_i = v2k(tgt)
        mid = (lo >> 1) + (hi >> 1) + (lo & hi & 1)
        cand_i = jnp.where(it % 4 == 3, mid, cand_i)
        cand = jnp.clip(cand_i, lo + 1, hi - 1)
        cnt = jnp.sum((key >= cand).astype(jnp.float32), axis=1, keepdims=True)
        ge = cnt >= kf
        upd_lo = act & ge
        upd_hi = act & ~ge
        wl_n = jnp.where(upd_hi, wl * 0.5, 1.0)
        wh_n = jnp.where(upd_lo, wh * 0.5, 1.0)
        lo_n = jnp.where(upd_lo, cand, lo)
        cl_n = jnp.where(upd_lo, cnt, cl)
        hi_n = jnp.where(upd_hi, cand, hi)
        ch_n = jnp.where(upd_hi, cnt, ch)
        return (lo_n, hi_n, cl_n, ch_n, wl_n, wh_n, it + 1)

    lo, hi, cl, ch, wl, wh, _ = jax.lax.while_loop(
        cond, body, (lo, hi, cl, ch, wl, wh, jnp.int32(0)))
    mask = key >= lo

    z = jnp.where(mask, g * jax.nn.sigmoid(g) * u, 0.0)
    o_ref[...] = jnp.dot(z.astype(jnp.bfloat16), wd_ref[...],
                         preferred_element_type=jnp.float32)


def kernel(x, w_gate, w_up, w_down):
    orig_shape = x.shape
    x2 = x.reshape(-1, _D_MODEL)
    n = x2.shape[0]
    wu = w_up.astype(jnp.bfloat16)
    wd = w_down.astype(jnp.bfloat16)
    out = pl.pallas_call(
        _ffn_kernel,
        grid=(n // _BLK,),
        in_specs=[
            pl.BlockSpec((_BLK, _D_MODEL), lambda i: (i, 0)),
            pl.BlockSpec((_D_MODEL, _D_FFN), lambda i: (0, 0)),
            pl.BlockSpec((_D_MODEL, _D_FFN), lambda i: (0, 0)),
            pl.BlockSpec((_D_FFN, _D_MODEL), lambda i: (0, 0)),
        ],
        out_specs=pl.BlockSpec((_BLK, _D_MODEL), lambda i: (i, 0)),
        out_shape=jax.ShapeDtypeStruct((n, _D_MODEL), jnp.float32),
        compiler_params=pltpu.CompilerParams(
            dimension_semantics=("arbitrary",),
        ),
    )(x2, w_gate, wu, wd)
    return out.reshape(orig_shape)


# speedup vs baseline: 1.1333x; 1.1333x over previous
import jax
import jax.numpy as jnp
from jax.experimental import pallas as pl
from jax.experimental.pallas import tpu as pltpu

_D_MODEL = 1024
_D_FFN = 4096
_TOP_K = 256
_BLK = 256


def _ffn_kernel(x_ref, wg_ref, wu_ref, wd_ref, o_ref):
    x = x_ref[...]
    g = jnp.dot(x, wg_ref[...], preferred_element_type=jnp.float32)
    u = jnp.dot(x.astype(jnp.bfloat16), wu_ref[...],
                preferred_element_type=jnp.float32)

    bits = jax.lax.bitcast_convert_type(g, jnp.int32)
    key = bits ^ ((bits >> 31) & jnp.int32(0x7FFFFFFF))

    cnt_pos = jnp.sum(key >= 0, axis=1, keepdims=True, dtype=jnp.int32)
    t = jnp.where(cnt_pos >= _TOP_K, jnp.int32(0), jnp.int32(-(2**31)))
    for b in range(30, -1, -1):
        cand = t | jnp.int32(1 << b)
        cnt = jnp.sum(key >= cand, axis=1, keepdims=True, dtype=jnp.int32)
        t = jnp.where(cnt >= _TOP_K, cand, t)
    mask = key >= t

    z = jnp.where(mask, g * jax.nn.sigmoid(g) * u, 0.0)
    o_ref[...] = jnp.dot(z.astype(jnp.bfloat16), wd_ref[...],
                         preferred_element_type=jnp.float32)


def kernel(x, w_gate, w_up, w_down):
    orig_shape = x.shape
    x2 = x.reshape(-1, _D_MODEL)
    n = x2.shape[0]
    wu = w_up.astype(jnp.bfloat16)
    wd = w_down.astype(jnp.bfloat16)
    out = pl.pallas_call(
        _ffn_kernel,
        grid=(n // _BLK,),
        in_specs=[
            pl.BlockSpec((_BLK, _D_MODEL), lambda i: (i, 0)),
            pl.BlockSpec((_D_MODEL, _D_FFN), lambda i: (0, 0)),
            pl.BlockSpec((_D_MODEL, _D_FFN), lambda i: (0, 0)),
            pl.BlockSpec((_D_FFN, _D_MODEL), lambda i: (0, 0)),
        ],
        out_specs=pl.BlockSpec((_BLK, _D_MODEL), lambda i: (i, 0)),
        out_shape=jax.ShapeDtypeStruct((n, _D_MODEL), jnp.float32),
        compiler_params=pltpu.CompilerParams(
            dimension_semantics=("arbitrary",),
        ),
    )(x2, w_gate, wu, wd)
    return out.reshape(orig_shape)
